# Initial kernel scaffold; baseline (speedup 1.0000x reference)
import jax
import jax.numpy as jnp
from jax.experimental import pallas as pl


def kernel(timestamps, event_features, cur_router_W, cur_router_b, cur_omega, cur_phase, prev_router_W, prev_router_b, prev_omega, prev_phase, kan_gamma, kan_beta, kan_spline_W, A_log, Bm, Cm, D_skip, dt_W, dt_b, gate_W, out_W):
    raise NotImplementedError("write your pallas kernel here")


# speedup vs baseline: 20.5486x; 20.5486x over previous
import functools

import jax
import jax.numpy as jnp
from jax.experimental import pallas as pl
from jax.experimental.pallas import tpu as pltpu

B, S, DT = 4, 2048, 128
FEAT = 64
E = 16
G = 8
NS = 16
GRID_MIN, GRID_MAX = -2.0, 2.0
INV_DENOM = 0.5
BL = B * S

L = 256
NB = S // L
ROWS = B * L

_HIGH = jax.lax.Precision.HIGHEST


def _dot(a, b):
    return jax.lax.dot_general(a, b, (((1,), (0,)), ((), ())),
                               precision=_HIGH,
                               preferred_element_type=jnp.float32)


def _route(f, t16, rW_ref, rb_ref, om_ref, ph_ref):
    cat = jnp.concatenate([t16[:, 0:1], f], axis=1)
    logits = jax.lax.dot_general(
        cat.astype(jnp.bfloat16), rW_ref[...].astype(jnp.bfloat16),
        (((1,), (0,)), ((), ())),
        preferred_element_type=jnp.float32) + rb_ref[0:1, :]
    iota = jax.lax.broadcasted_iota(jnp.int32, logits.shape, 1)
    v0 = jnp.max(logits, axis=1, keepdims=True)
    idx0 = jnp.min(jnp.where(logits == v0, iota, E), axis=1, keepdims=True)
    oh0 = (iota == idx0).astype(jnp.float32)
    l2 = jnp.where(iota == idx0, -jnp.inf, logits)
    v1 = jnp.max(l2, axis=1, keepdims=True)
    idx1 = jnp.min(jnp.where(l2 == v1, iota, E), axis=1, keepdims=True)
    oh1 = (iota == idx1).astype(jnp.float32)
    w0 = 1.0 / (1.0 + jnp.exp(v1 - v0))
    w1 = 1.0 - w0
    om = om_ref[...]
    ph = ph_ref[...]
    arg0 = _dot(oh0 * t16, om) + _dot(oh0, ph)
    arg1 = _dot(oh1 * t16, om) + _dot(oh1, ph)
    emb = w0 * jnp.sin(arg0) + w1 * jnp.sin(arg1)
    return emb, oh0 * w0 + oh1 * w1, oh0 + oh1


def _kern(t16_ref, pt16_ref, gaps_ref, feat_ref,
          crW_ref, crb_ref, com_ref, cph_ref,
          prW_ref, prb_ref, pom_ref, pph_ref,
          gam_ref, bet_ref, Wg_ref,
          AT_ref, BT_ref, CT_ref, Dsk_ref,
          dtW_ref, dtb_ref, gW_ref, oW_ref,
          out_ref, w_ref, m_ref,
          del_s, u_s, ys_s, h_s):
    ci = pl.program_id(0)
    f = feat_ref[...]
    t16 = t16_ref[...]

    emb_c, w_full, m_full = _route(f, t16, crW_ref, crb_ref, com_ref, cph_ref)
    w_ref[...] = w_full
    m_ref[...] = m_full
    emb_p, _, _ = _route(f, pt16_ref[...], prW_ref, prb_ref, pom_ref, pph_ref)

    d = emb_c - emb_p
    mu = jnp.mean(d, axis=1, keepdims=True)
    var = jnp.mean((d - mu) ** 2, axis=1, keepdims=True)
    h = (d - mu) / jnp.sqrt(var + 1e-5) * gam_ref[0:1, :] + bet_ref[0:1, :]

    kan = jnp.zeros_like(h)
    for g in range(G):
        gv = GRID_MIN + g * (GRID_MAX - GRID_MIN) / (G - 1)
        basis_g = 1.0 - jnp.tanh((h - gv) * INV_DENOM) ** 2
        kan = kan + _dot(basis_g, Wg_ref[g])

    z = _dot(kan, dtW_ref[...]) + dtb_ref[0:1, :]
    sp = jnp.maximum(z, 0.0) + jnp.log(1.0 + jnp.exp(-jnp.abs(z)))
    delta = sp * gaps_ref[...]
    del_s[...] = delta.reshape(L, B, DT)
    u_s[...] = (delta * kan).reshape(L, B, DT)

    AT = -jnp.exp(AT_ref[...])
    BT = BT_ref[...]
    CT = CT_ref[...]

    @pl.when(ci == 0)
    def _():
        h_s[...] = jnp.zeros_like(h_s)

    def body(s, hc):
        dt_row = del_s[pl.ds(s, 1), :, :]
        u_row = u_s[pl.ds(s, 1), :, :]
        dA = jnp.exp(dt_row.reshape(B, 1, DT) * AT[None, :, :])
        hc = hc * dA + u_row.reshape(B, 1, DT) * BT[None, :, :]
        y = jnp.sum(hc * CT[None, :, :], axis=1)
        ys_s[pl.ds(s, 1), :, :] = y.reshape(1, B, DT)
        return hc

    hc = jax.lax.fori_loop(0, L, body, h_s[...])
    h_s[...] = hc

    ys = ys_s[...].reshape(ROWS, DT)
    gz = _dot(kan, gW_ref[...])
    y = (ys + kan * Dsk_ref[0:1, :]) * (gz / (1.0 + jnp.exp(-gz)))
    out_ref[...] = _dot(y, oW_ref[...])


@functools.partial(jax.jit, static_argnames=("interpret",))
def _run(t16, pt16, gaps128, feat_s,
         cur_router_W, crb, cur_omega, cur_phase,
         prev_router_W, prb, prev_omega, prev_phase,
         gam, bet, Wg, AT_log, BT, CT, Dsk, dt_W, dtb, gate_W, out_W,
         interpret=False):
    tok_spec = lambda w: pl.BlockSpec((ROWS, w), lambda i: (i, 0))
    full = lambda a: pl.BlockSpec(a.shape, lambda i: tuple(0 for _ in a.shape))
    out_shapes = (
        jax.ShapeDtypeStruct((BL, DT), jnp.float32),
        jax.ShapeDtypeStruct((BL, E), jnp.float32),
        jax.ShapeDtypeStruct((BL, E), jnp.float32),
    )
    consts = (cur_router_W, crb, cur_omega, cur_phase,
              prev_router_W, prb, prev_omega, prev_phase,
              gam, bet, Wg, AT_log, BT, CT, Dsk, dt_W, dtb, gate_W, out_W)
    return pl.pallas_call(
        _kern,
        grid=(NB,),
        in_specs=[tok_spec(E), tok_spec(E), tok_spec(DT), tok_spec(FEAT)]
                 + [full(c) for c in consts],
        out_specs=[tok_spec(DT), tok_spec(E), tok_spec(E)],
        out_shape=out_shapes,
        scratch_shapes=[
            pltpu.VMEM((L, B, DT), jnp.float32),
            pltpu.VMEM((L, B, DT), jnp.float32),
            pltpu.VMEM((L, B, DT), jnp.float32),
            pltpu.VMEM((B, NS, DT), jnp.float32),
        ],
        compiler_params=pltpu.CompilerParams(
            dimension_semantics=("arbitrary",)),
        interpret=interpret,
    )(t16, pt16, gaps128, feat_s, *consts)


def kernel(timestamps, event_features, cur_router_W, cur_router_b, cur_omega,
           cur_phase, prev_router_W, prev_router_b, prev_omega, prev_phase,
           kan_gamma, kan_beta, kan_spline_W,
           A_log, Bm, Cm, D_skip, dt_W, dt_b, gate_W, out_W,
           interpret=False):
    ts = timestamps[:, :, 0]
    pts = jnp.concatenate([ts[:, :1], ts[:, :-1]], axis=1)
    gaps = jnp.concatenate(
        [jnp.full((B, 1), 1e-3, jnp.float32), ts[:, 1:] - ts[:, :-1]], axis=1)
    t_col = ts.T.reshape(BL, 1)
    pt_col = pts.T.reshape(BL, 1)
    g_col = gaps.T.reshape(BL, 1)
    t16 = jnp.broadcast_to(t_col, (BL, E))
    pt16 = jnp.broadcast_to(pt_col, (BL, E))
    gaps128 = jnp.broadcast_to(g_col, (BL, DT))
    feat_s = jnp.swapaxes(event_features, 0, 1).reshape(BL, FEAT)

    Wg = kan_spline_W.reshape(DT, G, DT).transpose(1, 0, 2)
    out_s, w_s, m_s = _run(
        t16, pt16, gaps128, feat_s,
        cur_router_W, cur_router_b.reshape(1, E), cur_omega, cur_phase,
        prev_router_W, prev_router_b.reshape(1, E), prev_omega, prev_phase,
        kan_gamma.reshape(1, DT), kan_beta.reshape(1, DT), Wg,
        A_log.T, Bm.T, Cm.T, D_skip.reshape(1, DT),
        dt_W, dt_b.reshape(1, DT), gate_W, out_W,
        interpret=interpret)

    out = jnp.swapaxes(out_s.reshape(S, B, DT), 0, 1)
    cur_w = jnp.swapaxes(w_s.reshape(S, B, E), 0, 1)
    cur_m = jnp.swapaxes(m_s.reshape(S, B, E), 0, 1)
    return out, cur_w, cur_m


# speedup vs baseline: 21.3138x; 1.0372x over previous
import functools

import jax
import jax.numpy as jnp
from jax.experimental import pallas as pl
from jax.experimental.pallas import tpu as pltpu

B, S, DT = 4, 2048, 128
FEAT = 64
E = 16
G = 8
NS = 16
GRID_MIN, GRID_MAX = -2.0, 2.0
INV_DENOM = 0.5
BL = B * S

L = 256
NB = S // L
ROWS = B * L

_HIGH = jax.lax.Precision.HIGHEST


def _dot(a, b):
    return jax.lax.dot_general(a, b, (((1,), (0,)), ((), ())),
                               precision=_HIGH,
                               preferred_element_type=jnp.float32)


def _route(f, t16, rW_ref, rb_ref, om_ref, ph_ref):
    cat = jnp.concatenate([t16[:, 0:1], f], axis=1)
    logits = jax.lax.dot_general(
        cat.astype(jnp.bfloat16), rW_ref[...].astype(jnp.bfloat16),
        (((1,), (0,)), ((), ())),
        preferred_element_type=jnp.float32) + rb_ref[0:1, :]
    iota = jax.lax.broadcasted_iota(jnp.int32, logits.shape, 1)
    v0 = jnp.max(logits, axis=1, keepdims=True)
    idx0 = jnp.min(jnp.where(logits == v0, iota, E), axis=1, keepdims=True)
    oh0 = (iota == idx0).astype(jnp.float32)
    l2 = jnp.where(iota == idx0, -jnp.inf, logits)
    v1 = jnp.max(l2, axis=1, keepdims=True)
    idx1 = jnp.min(jnp.where(l2 == v1, iota, E), axis=1, keepdims=True)
    oh1 = (iota == idx1).astype(jnp.float32)
    w0 = 1.0 / (1.0 + jnp.exp(v1 - v0))
    w1 = 1.0 - w0
    om = om_ref[...]
    ph = ph_ref[...]
    arg0 = _dot(oh0 * t16, om) + _dot(oh0, ph)
    arg1 = _dot(oh1 * t16, om) + _dot(oh1, ph)
    emb = w0 * jnp.sin(arg0) + w1 * jnp.sin(arg1)
    return emb, oh0 * w0 + oh1 * w1, oh0 + oh1


def _kern(t16_ref, pt16_ref, gaps_ref, feat_ref,
          crW_ref, crb_ref, com_ref, cph_ref,
          prW_ref, prb_ref, pom_ref, pph_ref,
          gam_ref, bet_ref, Wg_ref,
          AT_ref, BT_ref, CT_ref, Dsk_ref,
          dtW_ref, dtb_ref, gW_ref, oW_ref,
          out_ref, w_ref, m_ref,
          dA_s, u_s, ys_s, h_s):
    ci = pl.program_id(0)
    f = feat_ref[...]
    t16 = t16_ref[...]

    emb_c, w_full, m_full = _route(f, t16, crW_ref, crb_ref, com_ref, cph_ref)
    w_ref[...] = w_full
    m_ref[...] = m_full
    emb_p, _, _ = _route(f, pt16_ref[...], prW_ref, prb_ref, pom_ref, pph_ref)

    d = emb_c - emb_p
    mu = jnp.mean(d, axis=1, keepdims=True)
    var = jnp.mean((d - mu) ** 2, axis=1, keepdims=True)
    h = (d - mu) / jnp.sqrt(var + 1e-5) * gam_ref[0:1, :] + bet_ref[0:1, :]

    kan = jnp.zeros_like(h)
    for g in range(G):
        gv = GRID_MIN + g * (GRID_MAX - GRID_MIN) / (G - 1)
        basis_g = 1.0 - jnp.tanh((h - gv) * INV_DENOM) ** 2
        kan = kan + _dot(basis_g, Wg_ref[g])

    z = _dot(kan, dtW_ref[...]) + dtb_ref[0:1, :]
    sp = jnp.maximum(z, 0.0) + jnp.log(1.0 + jnp.exp(-jnp.abs(z)))
    delta = sp * gaps_ref[...]

    AT = -jnp.exp(AT_ref[...])
    CT = CT_ref[...]

    dA_s[...] = jnp.exp(delta.reshape(L, B, 1, DT) * AT[None, None, :, :])
    u_s[...] = ((delta * kan).reshape(L, B, 1, DT)
                * BT_ref[...][None, None, :, :])

    @pl.when(ci == 0)
    def _():
        h_s[...] = jnp.zeros_like(h_s)

    def body(s, hc):
        dA = dA_s[pl.ds(s, 1)].reshape(B, NS, DT)
        uB = u_s[pl.ds(s, 1)].reshape(B, NS, DT)
        hc = hc * dA + uB
        y = jnp.sum(hc * CT[None, :, :], axis=1)
        ys_s[pl.ds(s, 1), :, :] = y.reshape(1, B, DT)
        return hc

    hc = jax.lax.fori_loop(0, L, body, h_s[...])
    h_s[...] = hc

    ys = ys_s[...].reshape(ROWS, DT)
    gz = _dot(kan, gW_ref[...])
    y = (ys + kan * Dsk_ref[0:1, :]) * (gz / (1.0 + jnp.exp(-gz)))
    out_ref[...] = _dot(y, oW_ref[...])


@functools.partial(jax.jit, static_argnames=("interpret",))
def _run(t16, pt16, gaps128, feat_s,
         cur_router_W, crb, cur_omega, cur_phase,
         prev_router_W, prb, prev_omega, prev_phase,
         gam, bet, Wg, AT_log, BT, CT, Dsk, dt_W, dtb, gate_W, out_W,
         interpret=False):
    tok_spec = lambda w: pl.BlockSpec((ROWS, w), lambda i: (i, 0))
    full = lambda a: pl.BlockSpec(a.shape, lambda i: tuple(0 for _ in a.shape))
    out_shapes = (
        jax.ShapeDtypeStruct((BL, DT), jnp.float32),
        jax.ShapeDtypeStruct((BL, E), jnp.float32),
        jax.ShapeDtypeStruct((BL, E), jnp.float32),
    )
    consts = (cur_router_W, crb, cur_omega, cur_phase,
              prev_router_W, prb, prev_omega, prev_phase,
              gam, bet, Wg, AT_log, BT, CT, Dsk, dt_W, dtb, gate_W, out_W)
    return pl.pallas_call(
        _kern,
        grid=(NB,),
        in_specs=[tok_spec(E), tok_spec(E), tok_spec(DT), tok_spec(FEAT)]
                 + [full(c) for c in consts],
        out_specs=[tok_spec(DT), tok_spec(E), tok_spec(E)],
        out_shape=out_shapes,
        scratch_shapes=[
            pltpu.VMEM((L, B, NS, DT), jnp.float32),
            pltpu.VMEM((L, B, NS, DT), jnp.float32),
            pltpu.VMEM((L, B, DT), jnp.float32),
            pltpu.VMEM((B, NS, DT), jnp.float32),
        ],
        compiler_params=pltpu.CompilerParams(
            dimension_semantics=("arbitrary",)),
        interpret=interpret,
    )(t16, pt16, gaps128, feat_s, *consts)


def kernel(timestamps, event_features, cur_router_W, cur_router_b, cur_omega,
           cur_phase, prev_router_W, prev_router_b, prev_omega, prev_phase,
           kan_gamma, kan_beta, kan_spline_W,
           A_log, Bm, Cm, D_skip, dt_W, dt_b, gate_W, out_W,
           interpret=False):
    ts = timestamps[:, :, 0]
    pts = jnp.concatenate([ts[:, :1], ts[:, :-1]], axis=1)
    gaps = jnp.concatenate(
        [jnp.full((B, 1), 1e-3, jnp.float32), ts[:, 1:] - ts[:, :-1]], axis=1)
    t_col = ts.T.reshape(BL, 1)
    pt_col = pts.T.reshape(BL, 1)
    g_col = gaps.T.reshape(BL, 1)
    t16 = jnp.broadcast_to(t_col, (BL, E))
    pt16 = jnp.broadcast_to(pt_col, (BL, E))
    gaps128 = jnp.broadcast_to(g_col, (BL, DT))
    feat_s = jnp.swapaxes(event_features, 0, 1).reshape(BL, FEAT)

    Wg = kan_spline_W.reshape(DT, G, DT).transpose(1, 0, 2)
    out_s, w_s, m_s = _run(
        t16, pt16, gaps128, feat_s,
        cur_router_W, cur_router_b.reshape(1, E), cur_omega, cur_phase,
        prev_router_W, prev_router_b.reshape(1, E), prev_omega, prev_phase,
        kan_gamma.reshape(1, DT), kan_beta.reshape(1, DT), Wg,
        A_log.T, Bm.T, Cm.T, D_skip.reshape(1, DT),
        dt_W, dt_b.reshape(1, DT), gate_W, out_W,
        interpret=interpret)

    out = jnp.swapaxes(out_s.reshape(S, B, DT), 0, 1)
    cur_w = jnp.swapaxes(w_s.reshape(S, B, E), 0, 1)
    cur_m = jnp.swapaxes(m_s.reshape(S, B, E), 0, 1)
    return out, cur_w, cur_m
